# baseline (speedup 1.0000x reference)
import jax
import jax.numpy as jnp
from jax import lax
from jax.experimental import pallas as pl
from jax.experimental.pallas import tpu as pltpu
from jax.experimental.pallas import tpu_sc as plsc

NC = 2
NS = 16
NW = NC * NS

V = 1000000
B = 4096
L = 200
D = 32

B_PER_W = B // NW
CHUNK = 4
ROWS_PER_CHUNK = CHUNK * L
N_CHUNKS = B_PER_W // CHUNK
IDX_SEG = 80
SEGS_PER_CHUNK = ROWS_PER_CHUNK // IDX_SEG
IDX_ROWS = N_CHUNKS * SEGS_PER_CHUNK
UNROLL = 8
INNER_ITERS = L // UNROLL
F16_FIX = jnp.float32(2.0 ** 112)

TC_BLK = 8192
LOG_BLK = 13
TC_GRID = (V + TC_BLK - 1) // TC_BLK
HALF = D // 2
GROUPS = 128 // HALF
GRP_ROWS = TC_BLK // GROUPS
LOG_GRP = 10
OUT_ROWS = TC_GRID * GRP_ROWS
V_PAD = OUT_ROWS * GROUPS


def _tc_convert_body(t_ref, out_ref):
  t = jnp.maximum(t_ref[...], 0.0)
  y = t * jnp.float32(2.0 ** -112)
  u = lax.bitcast_convert_type(y, jnp.uint32)
  h = (u + jnp.uint32(0xFFF) + ((u >> 13) & jnp.uint32(1))) >> 13
  s = h[0:HALF, :] | (h[HALF:D, :] << 16)
  stacked = jnp.concatenate(
      [s[:, k * GRP_ROWS:(k + 1) * GRP_ROWS] for k in range(GROUPS)], axis=0)
  out_ref[...] = stacked.T


def _bf16_round(v):
  u = plsc.bitcast(v, jnp.uint32)
  r = (u + jnp.uint32(0x7FFF) + ((u >> 16) & jnp.uint32(1))) \
      & jnp.uint32(0xFFFF0000)
  return plsc.bitcast(r, jnp.float32)


def _sc_body(x_hbm, table_hbm, params_hbm, out_hbm,
             idx_v, rows0, rows1, params_v, outbuf, sem0, sem1):
  wid = lax.axis_index("s") * NC + lax.axis_index("c")

  pltpu.sync_copy(x_hbm.at[wid], idx_v)
  pltpu.sync_copy(params_hbm, params_v)

  def permute_row(j, _):
    for c in range(IDX_SEG // 16):
      e = idx_v[j, pl.ds(c * 16, 16)]
      i = e >> LOG_BLK
      o = e & (GRP_ROWS - 1)
      k = (e >> LOG_GRP) & (GROUPS - 1)
      idx_v[j, pl.ds(c * 16, 16)] = (((i << LOG_GRP) + o) << 3) + k
    return 0

  lax.fori_loop(0, IDX_ROWS, permute_row, 0)

  w0 = _bf16_round(params_v[pl.ds(0, 16)])
  w1 = _bf16_round(params_v[pl.ds(16, 16)])
  bias = params_v[pl.ds(32, 16)][0]
  lane_iota = lax.iota(jnp.int32, 16)

  zero16 = jnp.zeros((16,), jnp.float32)
  for i in range(B_PER_W // 16):
    outbuf[pl.ds(i * 16, 16)] = zero16

  bufs = (rows0, rows1)
  sems = (sem0, sem1)

  def issue_chunk(c, buf, sem):
    for j in range(SEGS_PER_CHUNK):
      pltpu.make_async_copy(
          table_hbm.at[idx_v.at[c * SEGS_PER_CHUNK + j]],
          buf.at[pl.ds(j * IDX_SEG, IDX_SEG)],
          sem,
      ).start()

  def drain_chunk(buf, sem):
    pltpu.make_async_copy(
        table_hbm.at[pl.ds(0, ROWS_PER_CHUNK)], buf, sem
    ).wait()

  def compute_chunk(c, buf):
    def sample_body(s, _):
      base = s * L

      def inner(k, acc):
        a0, a1, b0, b1 = acc
        r = base + k * UNROLL
        for u in range(UNROLL):
          v = buf[r + u]
          lo_b = (v << 16) >> 3
          hi_b = (v >> 3) & jnp.uint32(0x0FFFE000)
          lo = plsc.bitcast(lo_b, jnp.float32) * F16_FIX
          hi = plsc.bitcast(hi_b, jnp.float32) * F16_FIX
          if u % 2 == 0:
            a0 = a0 + lo
            a1 = a1 + hi
          else:
            b0 = b0 + lo
            b1 = b1 + hi
        return (a0, a1, b0, b1)

      zero = jnp.zeros((16,), jnp.float32)
      a0, a1, b0, b1 = lax.fori_loop(
          0, INNER_ITERS, inner, (zero, zero, zero, zero))
      a0 = a0 + b0
      a1 = a1 + b1
      m0 = _bf16_round(a0 * (1.0 / L))
      m1 = _bf16_round(a1 * (1.0 / L))
      dot = jnp.sum(m0 * w0 + m1 * w1, axis=0)
      res = jnp.maximum(dot + bias, 0.0)
      g = c * CHUNK + s
      grp = (g // 16) * 16
      val = jnp.where(lane_iota == (g - grp), res, 0.0)
      plsc.addupdate(outbuf.at[pl.ds(grp, 16)], val)
      return 0

    lax.fori_loop(0, CHUNK, sample_body, 0)

  issue_chunk(0, bufs[0], sems[0])
  issue_chunk(1, bufs[1], sems[1])

  def outer(i, _):
    for bsel in range(2):
      c = 2 * i + bsel
      drain_chunk(bufs[bsel], sems[bsel])
      compute_chunk(c, bufs[bsel])

      @pl.when(c + 2 < N_CHUNKS)
      def _():
        issue_chunk(c + 2, bufs[bsel], sems[bsel])

    return 0

  lax.fori_loop(0, N_CHUNKS // 2, outer, 0)

  pltpu.sync_copy(outbuf, out_hbm.at[pl.ds(wid * B_PER_W, B_PER_W)])


@jax.jit
def _run(x3, tT, params):
  packed = pl.pallas_call(
      _tc_convert_body,
      grid=(TC_GRID,),
      in_specs=[pl.BlockSpec((D, TC_BLK), lambda i: (0, i))],
      out_specs=pl.BlockSpec((GRP_ROWS, 128), lambda i: (i, 0)),
      out_shape=jax.ShapeDtypeStruct((OUT_ROWS, 128), jnp.uint32),
  )(tT)
  table_u = packed.reshape(V_PAD, HALF)

  mesh = plsc.VectorSubcoreMesh(core_axis_name="c", subcore_axis_name="s")
  kfn = pl.kernel(
      _sc_body,
      out_type=jax.ShapeDtypeStruct((B,), jnp.float32),
      mesh=mesh,
      scratch_types=[
          pltpu.VMEM((IDX_ROWS, IDX_SEG), jnp.int32),
          pltpu.VMEM((ROWS_PER_CHUNK, HALF), jnp.uint32),
          pltpu.VMEM((ROWS_PER_CHUNK, HALF), jnp.uint32),
          pltpu.VMEM((48,), jnp.float32),
          pltpu.VMEM((B_PER_W,), jnp.float32),
          pltpu.SemaphoreType.DMA,
          pltpu.SemaphoreType.DMA,
      ],
      compiler_params=pltpu.CompilerParams(
          needs_layout_passes=False, use_tc_tiling_on_sc=False),
  )
  return kfn(x3, table_u, params)


def kernel(x, table, W, b):
  x3 = x.astype(jnp.int32).reshape(NW, IDX_ROWS, IDX_SEG)
  params = jnp.concatenate(
      [W.reshape(D).astype(jnp.float32),
       jnp.broadcast_to(b.astype(jnp.float32), (16,))]
  )
  out = _run(x3, table.T, params)
  return out.reshape(B, 1)


# speedup vs baseline: 1.2221x; 1.2221x over previous
import jax
import jax.numpy as jnp
from jax import lax
from jax.experimental import pallas as pl
from jax.experimental.pallas import tpu as pltpu
from jax.experimental.pallas import tpu_sc as plsc

NC = 2
NS = 16
NW = NC * NS

V = 1000000
B = 4096
L = 200
D = 32

B_PER_W = B // NW
CHUNK = 4
ROWS_PER_CHUNK = CHUNK * L
N_CHUNKS = B_PER_W // CHUNK
IDX_SEG = 80
SEGS_PER_CHUNK = ROWS_PER_CHUNK // IDX_SEG
IDX_ROWS = N_CHUNKS * SEGS_PER_CHUNK
UNROLL = 8
INNER_ITERS = L // UNROLL
F16_FIX = jnp.float32(2.0 ** 112)

TC_BLK = 16384
LOG_BLK = 14
TC_GRID = (V + TC_BLK - 1) // TC_BLK
HALF = D // 2
GROUPS = 128 // HALF
GRP_ROWS = TC_BLK // GROUPS
LOG_GRP = 11
OUT_ROWS = TC_GRID * GRP_ROWS
V_PAD = OUT_ROWS * GROUPS


def _tc_convert_body(t_ref, out_ref):
  t = jnp.maximum(t_ref[...], 0.0)
  y = t * jnp.float32(2.0 ** -112)
  u = lax.bitcast_convert_type(y, jnp.uint32)
  h = (u + jnp.uint32(0xFFF) + ((u >> 13) & jnp.uint32(1))) >> 13
  s = h[0:HALF, :] | (h[HALF:D, :] << 16)
  stacked = jnp.concatenate(
      [s[:, k * GRP_ROWS:(k + 1) * GRP_ROWS] for k in range(GROUPS)], axis=0)
  out_ref[...] = stacked.T


def _bf16_round(v):
  u = plsc.bitcast(v, jnp.uint32)
  r = (u + jnp.uint32(0x7FFF) + ((u >> 16) & jnp.uint32(1))) \
      & jnp.uint32(0xFFFF0000)
  return plsc.bitcast(r, jnp.float32)


def _sc_body(x_hbm, table_hbm, params_hbm, out_hbm,
             idx_v, rows0, rows1, params_v, outbuf, sem0, sem1):
  wid = lax.axis_index("s") * NC + lax.axis_index("c")

  pltpu.sync_copy(x_hbm.at[wid], idx_v)
  pltpu.sync_copy(params_hbm, params_v)

  def permute_row(j, _):
    for c in range(IDX_SEG // 16):
      e = idx_v[j, pl.ds(c * 16, 16)]
      i = e >> LOG_BLK
      o = e & (GRP_ROWS - 1)
      k = (e >> LOG_GRP) & (GROUPS - 1)
      idx_v[j, pl.ds(c * 16, 16)] = (((i << LOG_GRP) + o) << 3) + k
    return 0

  lax.fori_loop(0, IDX_ROWS, permute_row, 0)

  w0 = _bf16_round(params_v[pl.ds(0, 16)])
  w1 = _bf16_round(params_v[pl.ds(16, 16)])
  bias = params_v[pl.ds(32, 16)][0]
  lane_iota = lax.iota(jnp.int32, 16)

  zero16 = jnp.zeros((16,), jnp.float32)
  for i in range(B_PER_W // 16):
    outbuf[pl.ds(i * 16, 16)] = zero16

  bufs = (rows0, rows1)
  sems = (sem0, sem1)

  def issue_chunk(c, buf, sem):
    for j in range(SEGS_PER_CHUNK):
      pltpu.make_async_copy(
          table_hbm.at[idx_v.at[c * SEGS_PER_CHUNK + j]],
          buf.at[pl.ds(j * IDX_SEG, IDX_SEG)],
          sem,
      ).start()

  def drain_chunk(buf, sem):
    pltpu.make_async_copy(
        table_hbm.at[pl.ds(0, ROWS_PER_CHUNK)], buf, sem
    ).wait()

  def compute_chunk(c, buf):
    def sample_body(s, _):
      base = s * L

      def inner(k, acc):
        a0, a1, b0, b1 = acc
        r = base + k * UNROLL
        for u in range(UNROLL):
          v = buf[r + u]
          lo_b = (v << 16) >> 3
          hi_b = (v >> 3) & jnp.uint32(0x0FFFE000)
          lo = plsc.bitcast(lo_b, jnp.float32)
          hi = plsc.bitcast(hi_b, jnp.float32)
          if u % 2 == 0:
            a0 = a0 + lo
            a1 = a1 + hi
          else:
            b0 = b0 + lo
            b1 = b1 + hi
        return (a0, a1, b0, b1)

      zero = jnp.zeros((16,), jnp.float32)
      a0, a1, b0, b1 = lax.fori_loop(
          0, INNER_ITERS, inner, (zero, zero, zero, zero))
      a0 = a0 + b0
      a1 = a1 + b1
      m0 = _bf16_round(a0 * (1.0 / L))
      m1 = _bf16_round(a1 * (1.0 / L))
      dot = jnp.sum(m0 * w0 + m1 * w1, axis=0)
      res = jnp.maximum(dot * F16_FIX + bias, 0.0)
      g = c * CHUNK + s
      grp = (g // 16) * 16
      val = jnp.where(lane_iota == (g - grp), res, 0.0)
      plsc.addupdate(outbuf.at[pl.ds(grp, 16)], val)
      return 0

    lax.fori_loop(0, CHUNK, sample_body, 0)

  issue_chunk(0, bufs[0], sems[0])
  issue_chunk(1, bufs[1], sems[1])

  def outer(i, _):
    for bsel in range(2):
      c = 2 * i + bsel
      drain_chunk(bufs[bsel], sems[bsel])
      compute_chunk(c, bufs[bsel])

      @pl.when(c + 2 < N_CHUNKS)
      def _():
        issue_chunk(c + 2, bufs[bsel], sems[bsel])

    return 0

  lax.fori_loop(0, N_CHUNKS // 2, outer, 0)

  pltpu.sync_copy(outbuf, out_hbm.at[pl.ds(wid * B_PER_W, B_PER_W)])


@jax.jit
def _run(x3, tT, params):
  packed = pl.pallas_call(
      _tc_convert_body,
      grid=(TC_GRID,),
      in_specs=[pl.BlockSpec((D, TC_BLK), lambda i: (0, i))],
      out_specs=pl.BlockSpec((GRP_ROWS, 128), lambda i: (i, 0)),
      out_shape=jax.ShapeDtypeStruct((OUT_ROWS, 128), jnp.uint32),
  )(tT)
  table_u = packed.reshape(V_PAD, HALF)

  mesh = plsc.VectorSubcoreMesh(core_axis_name="c", subcore_axis_name="s")
  kfn = pl.kernel(
      _sc_body,
      out_type=jax.ShapeDtypeStruct((B,), jnp.float32),
      mesh=mesh,
      scratch_types=[
          pltpu.VMEM((IDX_ROWS, IDX_SEG), jnp.int32),
          pltpu.VMEM((ROWS_PER_CHUNK, HALF), jnp.uint32),
          pltpu.VMEM((ROWS_PER_CHUNK, HALF), jnp.uint32),
          pltpu.VMEM((48,), jnp.float32),
          pltpu.VMEM((B_PER_W,), jnp.float32),
          pltpu.SemaphoreType.DMA,
          pltpu.SemaphoreType.DMA,
      ],
      compiler_params=pltpu.CompilerParams(
          needs_layout_passes=False, use_tc_tiling_on_sc=False),
  )
  return kfn(x3, table_u, params)


def kernel(x, table, W, b):
  x3 = x.astype(jnp.int32).reshape(NW, IDX_ROWS, IDX_SEG)
  params = jnp.concatenate(
      [W.reshape(D).astype(jnp.float32),
       jnp.broadcast_to(b.astype(jnp.float32), (16,))]
  )
  out = _run(x3, table.T, params)
  return out.reshape(B, 1)


# speedup vs baseline: 1.4046x; 1.1493x over previous
import jax
import jax.numpy as jnp
from jax import lax
from jax.experimental import pallas as pl
from jax.experimental.pallas import tpu as pltpu
from jax.experimental.pallas import tpu_sc as plsc

NC = 2
NS = 16
NW = NC * NS

V = 1000000
B = 4096
L = 200
D = 32

B_PER_W = B // NW
CHUNK = 8
ROWS_PER_CHUNK = CHUNK * L
N_CHUNKS = B_PER_W // CHUNK
IDX_SEG = 80
SEGS_PER_CHUNK = ROWS_PER_CHUNK // IDX_SEG
IDX_ROWS = N_CHUNKS * SEGS_PER_CHUNK
UNROLL = 8
INNER_ITERS = L // UNROLL
F16_FIX = jnp.float32(2.0 ** 112)

TC_BLK = 32768
LOG_BLK = 15
TC_GRID = (V + TC_BLK - 1) // TC_BLK
HALF = D // 2
GROUPS = 128 // HALF
GRP_ROWS = TC_BLK // GROUPS
LOG_GRP = 12
OUT_ROWS = TC_GRID * GRP_ROWS
V_PAD = OUT_ROWS * GROUPS


def _tc_convert_body(t_ref, out_ref):
  t = jnp.maximum(t_ref[...], 0.0)
  y = t * jnp.float32(2.0 ** -112)
  u = lax.bitcast_convert_type(y, jnp.uint32)
  h = (u + jnp.uint32(0xFFF) + ((u >> 13) & jnp.uint32(1))) >> 13
  s = h[0:HALF, :] | (h[HALF:D, :] << 16)
  stacked = jnp.concatenate(
      [s[:, k * GRP_ROWS:(k + 1) * GRP_ROWS] for k in range(GROUPS)], axis=0)
  out_ref[...] = stacked.T


def _bf16_round(v):
  u = plsc.bitcast(v, jnp.uint32)
  r = (u + jnp.uint32(0x7FFF) + ((u >> 16) & jnp.uint32(1))) \
      & jnp.uint32(0xFFFF0000)
  return plsc.bitcast(r, jnp.float32)


def _sc_body(x_hbm, table_hbm, params_hbm, out_hbm,
             idx_v, rows0, rows1, params_v, outbuf, sem0, sem1):
  wid = lax.axis_index("s") * NC + lax.axis_index("c")

  pltpu.sync_copy(x_hbm.at[wid], idx_v)
  pltpu.sync_copy(params_hbm, params_v)

  def permute_row(j, _):
    for c in range(IDX_SEG // 16):
      e = idx_v[j, pl.ds(c * 16, 16)]
      i = e >> LOG_BLK
      o = e & (GRP_ROWS - 1)
      k = (e >> LOG_GRP) & (GROUPS - 1)
      idx_v[j, pl.ds(c * 16, 16)] = (((i << LOG_GRP) + o) << 3) + k
    return 0

  lax.fori_loop(0, IDX_ROWS, permute_row, 0)

  w0 = _bf16_round(params_v[pl.ds(0, 16)])
  w1 = _bf16_round(params_v[pl.ds(16, 16)])
  bias = params_v[pl.ds(32, 16)][0]
  lane_iota = lax.iota(jnp.int32, 16)

  zero16 = jnp.zeros((16,), jnp.float32)
  for i in range(B_PER_W // 16):
    outbuf[pl.ds(i * 16, 16)] = zero16

  bufs = (rows0, rows1)
  sems = (sem0, sem1)

  def issue_chunk(c, buf, sem):
    for j in range(SEGS_PER_CHUNK):
      pltpu.make_async_copy(
          table_hbm.at[idx_v.at[c * SEGS_PER_CHUNK + j]],
          buf.at[pl.ds(j * IDX_SEG, IDX_SEG)],
          sem,
      ).start()

  def drain_chunk(buf, sem):
    pltpu.make_async_copy(
        table_hbm.at[pl.ds(0, ROWS_PER_CHUNK)], buf, sem
    ).wait()

  def compute_chunk(c, buf):
    def sample_body(s, _):
      base = s * L

      def inner(k, acc):
        a0, a1, b0, b1 = acc
        r = base + k * UNROLL
        for u in range(UNROLL):
          v = buf[r + u]
          lo_b = (v << 16) >> 3
          hi_b = (v >> 3) & jnp.uint32(0x0FFFE000)
          lo = plsc.bitcast(lo_b, jnp.float32)
          hi = plsc.bitcast(hi_b, jnp.float32)
          if u % 2 == 0:
            a0 = a0 + lo
            a1 = a1 + hi
          else:
            b0 = b0 + lo
            b1 = b1 + hi
        return (a0, a1, b0, b1)

      zero = jnp.zeros((16,), jnp.float32)
      a0, a1, b0, b1 = lax.fori_loop(
          0, INNER_ITERS, inner, (zero, zero, zero, zero))
      a0 = a0 + b0
      a1 = a1 + b1
      m0 = _bf16_round(a0 * (1.0 / L))
      m1 = _bf16_round(a1 * (1.0 / L))
      dot = jnp.sum(m0 * w0 + m1 * w1, axis=0)
      res = jnp.maximum(dot * F16_FIX + bias, 0.0)
      g = c * CHUNK + s
      grp = (g // 16) * 16
      val = jnp.where(lane_iota == (g - grp), res, 0.0)
      plsc.addupdate(outbuf.at[pl.ds(grp, 16)], val)
      return 0

    lax.fori_loop(0, CHUNK, sample_body, 0)

  issue_chunk(0, bufs[0], sems[0])
  issue_chunk(1, bufs[1], sems[1])

  def outer(i, _):
    for bsel in range(2):
      c = 2 * i + bsel
      drain_chunk(bufs[bsel], sems[bsel])
      compute_chunk(c, bufs[bsel])

      @pl.when(c + 2 < N_CHUNKS)
      def _():
        issue_chunk(c + 2, bufs[bsel], sems[bsel])

    return 0

  lax.fori_loop(0, N_CHUNKS // 2, outer, 0)

  pltpu.sync_copy(outbuf, out_hbm.at[pl.ds(wid * B_PER_W, B_PER_W)])


@jax.jit
def _run(x3, tT, params):
  packed = pl.pallas_call(
      _tc_convert_body,
      grid=(TC_GRID,),
      in_specs=[pl.BlockSpec((D, TC_BLK), lambda i: (0, i))],
      out_specs=pl.BlockSpec((GRP_ROWS, 128), lambda i: (i, 0)),
      out_shape=jax.ShapeDtypeStruct((OUT_ROWS, 128), jnp.uint32),
  )(tT)
  table_u = packed.reshape(V_PAD, HALF)

  mesh = plsc.VectorSubcoreMesh(core_axis_name="c", subcore_axis_name="s")
  kfn = pl.kernel(
      _sc_body,
      out_type=jax.ShapeDtypeStruct((B,), jnp.float32),
      mesh=mesh,
      scratch_types=[
          pltpu.VMEM((IDX_ROWS, IDX_SEG), jnp.int32),
          pltpu.VMEM((ROWS_PER_CHUNK, HALF), jnp.uint32),
          pltpu.VMEM((ROWS_PER_CHUNK, HALF), jnp.uint32),
          pltpu.VMEM((48,), jnp.float32),
          pltpu.VMEM((B_PER_W,), jnp.float32),
          pltpu.SemaphoreType.DMA,
          pltpu.SemaphoreType.DMA,
      ],
      compiler_params=pltpu.CompilerParams(
          needs_layout_passes=False, use_tc_tiling_on_sc=False),
  )
  return kfn(x3, table_u, params)


def kernel(x, table, W, b):
  x3 = x.astype(jnp.int32).reshape(NW, IDX_ROWS, IDX_SEG)
  params = jnp.concatenate(
      [W.reshape(D).astype(jnp.float32),
       jnp.broadcast_to(b.astype(jnp.float32), (16,))]
  )
  out = _run(x3, table.T, params)
  return out.reshape(B, 1)
